# baseline (speedup 1.0000x reference)
import functools

import jax
import jax.numpy as jnp
from jax import lax
from jax.experimental import pallas as pl
from jax.experimental.pallas import tpu as pltpu
from jax.experimental.pallas import tpu_sc as plsc

N = 10000
NPAD = 10240
PT = NPAD // 16
E = 320000
CHUNK = 128
NW = 32
CHPW = 80
EPAD = NW * CHPW * CHUNK
G = 64
IN_CH = 128
HID = 512
OUT_CH = 64


DEPTH = 5


def _make_sc_prop(split):
  C = 64
  CH = (EPAD // CHUNK) // 16 if split else CHPW
  mesh = plsc.VectorSubcoreMesh(core_axis_name="c", subcore_axis_name="s")
  if split:
    idx_shape = (2, 16, CH, CHUNK)
  else:
    idx_shape = (NW, CH, CHUNK)

  @functools.partial(
      pl.kernel,
      out_type=jax.ShapeDtypeStruct((2, NPAD, C), jnp.float32),
      mesh=mesh,
      scratch_types=[
          pltpu.VMEM_SHARED((NPAD, C), jnp.float32),
          pltpu.VMEM((CH, CHUNK), jnp.int32),
          pltpu.VMEM((CH, CHUNK), jnp.int32),
          [pltpu.VMEM((CHUNK, C), jnp.float32) for _ in range(DEPTH)],
          [pltpu.SemaphoreType.DMA for _ in range(DEPTH)],
          [pltpu.SemaphoreType.DMA for _ in range(DEPTH)],
      ],
      compiler_params=pltpu.CompilerParams(use_tc_tiling_on_sc=False),
  )
  def prop(u_hbm, src_hbm, dst_hbm, zeros_hbm, out, acc_sh,
           sidx_a, didx_a, rows, sg, ss):
    cid = lax.axis_index("c")
    sid = lax.axis_index("s")
    if split:
      pltpu.sync_copy(src_hbm.at[cid, sid], sidx_a)
      pltpu.sync_copy(dst_hbm.at[sid], didx_a)
    else:
      wid = cid * 16 + sid
      pltpu.sync_copy(src_hbm.at[wid], sidx_a)
      pltpu.sync_copy(dst_hbm.at[wid], didx_a)
    pltpu.sync_copy(zeros_hbm, acc_sh.at[pl.ds(sid * PT, PT)])
    plsc.subcore_barrier()

    for b in range(DEPTH - 1):
      pltpu.async_copy(u_hbm.at[sidx_a.at[b]], rows[b], sg[b])

    def body(i, carry):
      for b in range(DEPTH):
        k = i * DEPTH + b
        pltpu.make_async_copy(u_hbm.at[sidx_a.at[k]], rows[b], sg[b]).wait()

        bp = (b - 1) % DEPTH
        @pl.when(k >= 1)
        def _():
          pltpu.make_async_copy(
              rows[bp], acc_sh.at[didx_a.at[k]], ss[bp]).wait()

        @pl.when(k + DEPTH - 1 < CH)
        def _():
          pltpu.async_copy(u_hbm.at[sidx_a.at[k + DEPTH - 1]], rows[bp],
                           sg[bp])

        pltpu.async_copy(rows[b], acc_sh.at[didx_a.at[k]], ss[b], add=True)
      return carry

    lax.fori_loop(0, CH // DEPTH, body, 0)
    pltpu.make_async_copy(
        rows[(CH - 1) % DEPTH], acc_sh.at[didx_a.at[CH - 1]],
        ss[(CH - 1) % DEPTH]).wait()
    plsc.subcore_barrier()
    pltpu.sync_copy(acc_sh.at[pl.ds(sid * PT, PT)],
                    out.at[cid, pl.ds(sid * PT, PT)])

  return prop


def _make_sc_deg():
  mesh = plsc.VectorSubcoreMesh(core_axis_name="c", subcore_axis_name="s")
  C = 16

  @functools.partial(
      pl.kernel,
      out_type=jax.ShapeDtypeStruct((2, NPAD, C), jnp.float32),
      mesh=mesh,
      scratch_types=[
          pltpu.VMEM_SHARED((NPAD, C), jnp.float32),
          pltpu.VMEM((CHPW, CHUNK), jnp.int32),
          pltpu.VMEM((CHUNK, C), jnp.float32),
          pltpu.SemaphoreType.DMA,
      ],
      compiler_params=pltpu.CompilerParams(use_tc_tiling_on_sc=False),
  )
  def deg(dst3_hbm, ones_hbm, zeros_hbm, out, acc_sh, didx_a, ones_v, ss):
    cid = lax.axis_index("c")
    sid = lax.axis_index("s")
    wid = cid * 16 + sid
    pltpu.sync_copy(dst3_hbm.at[wid], didx_a)
    pltpu.sync_copy(zeros_hbm, acc_sh.at[pl.ds(sid * PT, PT)])
    pltpu.sync_copy(ones_hbm, ones_v)
    plsc.subcore_barrier()
    D = 8

    def body(k, carry):
      pltpu.async_copy(ones_v, acc_sh.at[didx_a.at[k]], ss, add=True)

      @pl.when(k >= D)
      def _():
        pltpu.make_async_copy(ones_v, acc_sh.at[didx_a.at[k]], ss).wait()

      return carry

    lax.fori_loop(0, CHPW, body, 0)
    for _ in range(D):
      pltpu.make_async_copy(ones_v, acc_sh.at[didx_a.at[0]], ss).wait()
    plsc.subcore_barrier()
    pltpu.sync_copy(acc_sh.at[pl.ds(sid * PT, PT)],
                    out.at[cid, pl.ds(sid * PT, PT)])

  return deg


_RB = 1024


def _tc_prescale_body(x_ref, d_ref, u_ref, dinv_ref):
  dinv = lax.rsqrt(1.0 + d_ref[0, :, :1] + d_ref[1, :, :1])
  u_ref[0] = x_ref[:, :64] * dinv
  u_ref[1] = x_ref[:, 64:] * dinv
  dinv_ref[...] = jnp.broadcast_to(dinv, dinv_ref.shape)


def _tc_layer1_body(u1_ref, a_ref, dinv_ref, w1_ref, b1_ref, w2_ref, u2_ref):
  i = pl.program_id(0)
  dinv = dinv_ref[:, :1]
  p1 = dinv * jnp.concatenate(
      [u1_ref[0] + a_ref[0], u1_ref[1] + a_ref[1]], axis=1)
  h1 = jnp.maximum(
      jnp.dot(p1, w1_ref[...], preferred_element_type=jnp.float32)
      + b1_ref[...], 0.0)
  t = jnp.dot(h1, w2_ref[...], preferred_element_type=jnp.float32)
  row = i * _RB + lax.broadcasted_iota(jnp.int32, (_RB, 1), 0)
  u2_ref[...] = jnp.where(row < N, dinv * t, 0.0)


def _tc_pool_body(u2_ref, c_ref, dinv_ref, b2_ref, batch_ref, o_ref, cnt_ref):
  i = pl.program_id(0)
  nsteps = pl.num_programs(0)
  dinv = dinv_ref[:, :1]
  p2 = dinv * (u2_ref[...] + c_ref[0] + c_ref[1])
  h2 = jnp.maximum(p2 + b2_ref[...], 0.0)
  row = i * _RB + lax.broadcasted_iota(jnp.int32, (_RB, 1), 0)
  h2 = jnp.where(row < N, h2, 0.0)
  m = (batch_ref[...] ==
       lax.broadcasted_iota(jnp.int32, (1, G), 1)).astype(jnp.float32)
  part = lax.dot_general(m, h2, (((0,), (0,)), ((), ())),
                         preferred_element_type=jnp.float32)
  pcnt = lax.dot_general(m, jnp.ones((_RB, 1), jnp.float32),
                         (((0,), (0,)), ((), ())),
                         preferred_element_type=jnp.float32)

  @pl.when(i == 0)
  def _():
    o_ref[...] = jnp.zeros_like(o_ref)
    cnt_ref[...] = jnp.zeros_like(cnt_ref)

  o_ref[...] += part
  cnt_ref[:, :1] += pcnt

  @pl.when(i == nsteps - 1)
  def _():
    o_ref[...] = o_ref[...] / jnp.maximum(cnt_ref[:, :1], 1.0)


def kernel(x, edge_index, batch, W1, b1, W2, b2):
  f32 = jnp.float32
  pad_e = EPAD - E
  pad_idx = N + jnp.arange(pad_e, dtype=jnp.int32) % (NPAD - N)
  srcf = jnp.concatenate([edge_index[0], pad_idx])
  dstf = jnp.concatenate([edge_index[1], pad_idx])
  src_p = srcf.reshape(NW, CHPW, CHUNK)
  dst_p = dstf.reshape(NW, CHPW, CHUNK)
  ch1 = (EPAD // CHUNK) // 16
  sl = srcf.reshape(16, ch1, CHUNK)
  src4 = jnp.stack([sl, sl + NPAD])
  dst4 = dstf.reshape(16, ch1, CHUNK)
  x_p = jnp.pad(x, ((0, NPAD - N), (0, 0)))
  batch_p = jnp.concatenate(
      [batch, jnp.full((NPAD - N,), G, jnp.int32)]).reshape(NPAD, 1)
  b1r = b1.reshape(1, HID)
  b2r = b2.reshape(1, OUT_CH)
  zeros16 = jnp.zeros((PT, 16), f32)
  zeros64 = jnp.zeros((PT, OUT_CH), f32)
  ones16 = jnp.ones((CHUNK, 16), f32)

  d = _make_sc_deg()(dst_p, ones16, zeros16)

  grid = NPAD // _RB
  u3, dinv16 = pl.pallas_call(
      _tc_prescale_body,
      grid=(grid,),
      in_specs=[
          pl.BlockSpec((_RB, IN_CH), lambda i: (i, 0)),
          pl.BlockSpec((2, _RB, 16), lambda i: (0, i, 0)),
      ],
      out_specs=(pl.BlockSpec((2, _RB, 64), lambda i: (0, i, 0)),
                 pl.BlockSpec((_RB, 16), lambda i: (i, 0))),
      out_shape=(jax.ShapeDtypeStruct((2, NPAD, 64), f32),
                 jax.ShapeDtypeStruct((NPAD, 16), f32)),
  )(x_p, d)

  a = _make_sc_prop(True)(u3.reshape(2 * NPAD, 64), src4, dst4, zeros64)

  u2 = pl.pallas_call(
      _tc_layer1_body,
      grid=(grid,),
      in_specs=[
          pl.BlockSpec((2, _RB, 64), lambda i: (0, i, 0)),
          pl.BlockSpec((2, _RB, 64), lambda i: (0, i, 0)),
          pl.BlockSpec((_RB, 16), lambda i: (i, 0)),
          pl.BlockSpec((IN_CH, HID), lambda i: (0, 0)),
          pl.BlockSpec((1, HID), lambda i: (0, 0)),
          pl.BlockSpec((HID, OUT_CH), lambda i: (0, 0)),
      ],
      out_specs=pl.BlockSpec((_RB, OUT_CH), lambda i: (i, 0)),
      out_shape=jax.ShapeDtypeStruct((NPAD, OUT_CH), f32),
  )(u3, a, dinv16, W1, b1r, W2)

  c = _make_sc_prop(False)(u2, src_p, dst_p, zeros64)

  out = pl.pallas_call(
      _tc_pool_body,
      grid=(grid,),
      in_specs=[
          pl.BlockSpec((_RB, OUT_CH), lambda i: (i, 0)),
          pl.BlockSpec((2, _RB, OUT_CH), lambda i: (0, i, 0)),
          pl.BlockSpec((_RB, 16), lambda i: (i, 0)),
          pl.BlockSpec((1, OUT_CH), lambda i: (0, 0)),
          pl.BlockSpec((_RB, 1), lambda i: (i, 0)),
      ],
      out_specs=pl.BlockSpec((G, OUT_CH), lambda i: (0, 0)),
      out_shape=jax.ShapeDtypeStruct((G, OUT_CH), f32),
      scratch_shapes=[pltpu.VMEM((G, 128), f32)],
  )(u2, c, dinv16, b2r, batch_p)
  return out


# speedup vs baseline: 1.0148x; 1.0148x over previous
import functools

import jax
import jax.numpy as jnp
from jax import lax
from jax.experimental import pallas as pl
from jax.experimental.pallas import tpu as pltpu
from jax.experimental.pallas import tpu_sc as plsc

N = 10000
NPAD = 10240
PT = NPAD // 16
E = 320000
CHUNK = 128
NW = 32
CHPW = 80
EPAD = NW * CHPW * CHUNK
G = 64
IN_CH = 128
HID = 512
OUT_CH = 64


DEPTH = 4


def _make_sc_prop(split):
  C = 64
  CH = (EPAD // CHUNK) // 16 if split else CHPW
  mesh = plsc.VectorSubcoreMesh(core_axis_name="c", subcore_axis_name="s")

  @functools.partial(
      pl.kernel,
      out_type=jax.ShapeDtypeStruct((2, NPAD, C), jnp.float32),
      mesh=mesh,
      scratch_types=[
          pltpu.VMEM_SHARED((NPAD, C), jnp.float32),
          pltpu.VMEM((CH, CHUNK), jnp.int32),
          pltpu.VMEM((CH, CHUNK), jnp.int32),
          [pltpu.VMEM((CHUNK, C), jnp.float32) for _ in range(DEPTH)],
          [pltpu.SemaphoreType.DMA for _ in range(DEPTH)],
          [pltpu.SemaphoreType.DMA for _ in range(DEPTH)],
      ],
      compiler_params=pltpu.CompilerParams(use_tc_tiling_on_sc=False),
  )
  def prop(u_hbm, src_hbm, dst_hbm, zeros_hbm, out, acc_sh,
           sidx_a, didx_a, rows, sg, ss):
    cid = lax.axis_index("c")
    sid = lax.axis_index("s")
    if split:
      u_view = u_hbm.at[cid]
      pltpu.sync_copy(src_hbm.at[sid], sidx_a)
      pltpu.sync_copy(dst_hbm.at[sid], didx_a)
    else:
      u_view = u_hbm
      wid = cid * 16 + sid
      pltpu.sync_copy(src_hbm.at[wid], sidx_a)
      pltpu.sync_copy(dst_hbm.at[wid], didx_a)
    pltpu.sync_copy(zeros_hbm, acc_sh.at[pl.ds(sid * PT, PT)])
    plsc.subcore_barrier()

    for b in range(DEPTH - 1):
      pltpu.async_copy(u_view.at[sidx_a.at[b]], rows[b], sg[b])

    def body(i, carry):
      for b in range(DEPTH):
        k = i * DEPTH + b
        pltpu.make_async_copy(u_view.at[sidx_a.at[k]], rows[b], sg[b]).wait()

        bp = (b - 1) % DEPTH
        @pl.when(k >= 1)
        def _():
          pltpu.make_async_copy(
              rows[bp], acc_sh.at[didx_a.at[k]], ss[bp]).wait()

        @pl.when(k + DEPTH - 1 < CH)
        def _():
          pltpu.async_copy(u_view.at[sidx_a.at[k + DEPTH - 1]], rows[bp],
                           sg[bp])

        pltpu.async_copy(rows[b], acc_sh.at[didx_a.at[k]], ss[b], add=True)
      return carry

    lax.fori_loop(0, CH // DEPTH, body, 0)
    pltpu.make_async_copy(
        rows[(CH - 1) % DEPTH], acc_sh.at[didx_a.at[CH - 1]],
        ss[(CH - 1) % DEPTH]).wait()
    plsc.subcore_barrier()
    pltpu.sync_copy(acc_sh.at[pl.ds(sid * PT, PT)],
                    out.at[cid, pl.ds(sid * PT, PT)])

  return prop


def _make_sc_deg():
  mesh = plsc.VectorSubcoreMesh(core_axis_name="c", subcore_axis_name="s")
  C = 16

  @functools.partial(
      pl.kernel,
      out_type=jax.ShapeDtypeStruct((2, NPAD, C), jnp.float32),
      mesh=mesh,
      scratch_types=[
          pltpu.VMEM_SHARED((NPAD, C), jnp.float32),
          pltpu.VMEM((CHPW, CHUNK), jnp.int32),
          pltpu.VMEM((CHUNK, C), jnp.float32),
          pltpu.SemaphoreType.DMA,
      ],
      compiler_params=pltpu.CompilerParams(use_tc_tiling_on_sc=False),
  )
  def deg(dst3_hbm, ones_hbm, zeros_hbm, out, acc_sh, didx_a, ones_v, ss):
    cid = lax.axis_index("c")
    sid = lax.axis_index("s")
    wid = cid * 16 + sid
    pltpu.sync_copy(dst3_hbm.at[wid], didx_a)
    pltpu.sync_copy(zeros_hbm, acc_sh.at[pl.ds(sid * PT, PT)])
    pltpu.sync_copy(ones_hbm, ones_v)
    plsc.subcore_barrier()
    D = 8

    def body(k, carry):
      pltpu.async_copy(ones_v, acc_sh.at[didx_a.at[k]], ss, add=True)

      @pl.when(k >= D)
      def _():
        pltpu.make_async_copy(ones_v, acc_sh.at[didx_a.at[k]], ss).wait()

      return carry

    lax.fori_loop(0, CHPW, body, 0)
    for _ in range(D):
      pltpu.make_async_copy(ones_v, acc_sh.at[didx_a.at[0]], ss).wait()
    plsc.subcore_barrier()
    pltpu.sync_copy(acc_sh.at[pl.ds(sid * PT, PT)],
                    out.at[cid, pl.ds(sid * PT, PT)])

  return deg


_RB = 1024


def _tc_prescale_body(x_ref, d_ref, u_ref, dinv_ref):
  dinv = lax.rsqrt(1.0 + d_ref[0, :, :1] + d_ref[1, :, :1])
  u_ref[0] = x_ref[:, :64] * dinv
  u_ref[1] = x_ref[:, 64:] * dinv
  dinv_ref[...] = jnp.broadcast_to(dinv, dinv_ref.shape)


def _tc_layer1_body(u1_ref, a_ref, dinv_ref, w1_ref, b1_ref, w2_ref, u2_ref):
  i = pl.program_id(0)
  dinv = dinv_ref[:, :1]
  p1 = dinv * jnp.concatenate(
      [u1_ref[0] + a_ref[0], u1_ref[1] + a_ref[1]], axis=1)
  h1 = jnp.maximum(
      jnp.dot(p1, w1_ref[...], preferred_element_type=jnp.float32)
      + b1_ref[...], 0.0)
  t = jnp.dot(h1, w2_ref[...], preferred_element_type=jnp.float32)
  row = i * _RB + lax.broadcasted_iota(jnp.int32, (_RB, 1), 0)
  u2_ref[...] = jnp.where(row < N, dinv * t, 0.0)


def _tc_pool_body(u2_ref, c_ref, dinv_ref, b2_ref, batch_ref, o_ref, cnt_ref):
  i = pl.program_id(0)
  nsteps = pl.num_programs(0)
  dinv = dinv_ref[:, :1]
  p2 = dinv * (u2_ref[...] + c_ref[0] + c_ref[1])
  h2 = jnp.maximum(p2 + b2_ref[...], 0.0)
  row = i * _RB + lax.broadcasted_iota(jnp.int32, (_RB, 1), 0)
  h2 = jnp.where(row < N, h2, 0.0)
  m = (batch_ref[...] ==
       lax.broadcasted_iota(jnp.int32, (1, G), 1)).astype(jnp.float32)
  part = lax.dot_general(m, h2, (((0,), (0,)), ((), ())),
                         preferred_element_type=jnp.float32)
  pcnt = lax.dot_general(m, jnp.ones((_RB, 1), jnp.float32),
                         (((0,), (0,)), ((), ())),
                         preferred_element_type=jnp.float32)

  @pl.when(i == 0)
  def _():
    o_ref[...] = jnp.zeros_like(o_ref)
    cnt_ref[...] = jnp.zeros_like(cnt_ref)

  o_ref[...] += part
  cnt_ref[:, :1] += pcnt

  @pl.when(i == nsteps - 1)
  def _():
    o_ref[...] = o_ref[...] / jnp.maximum(cnt_ref[:, :1], 1.0)


def kernel(x, edge_index, batch, W1, b1, W2, b2):
  f32 = jnp.float32
  pad_e = EPAD - E
  pad_idx = N + jnp.arange(pad_e, dtype=jnp.int32) % (NPAD - N)
  srcf = jnp.concatenate([edge_index[0], pad_idx])
  dstf = jnp.concatenate([edge_index[1], pad_idx])
  src_p = srcf.reshape(NW, CHPW, CHUNK)
  dst_p = dstf.reshape(NW, CHPW, CHUNK)
  ch1 = (EPAD // CHUNK) // 16
  src_sp = srcf.reshape(16, ch1, CHUNK)
  dst_sp = dstf.reshape(16, ch1, CHUNK)
  batch_p = jnp.concatenate(
      [batch, jnp.full((NPAD - N,), G, jnp.int32)]).reshape(NPAD, 1)
  b1r = b1.reshape(1, HID)
  b2r = b2.reshape(1, OUT_CH)
  zeros16 = jnp.zeros((PT, 16), f32)
  zeros64 = jnp.zeros((PT, OUT_CH), f32)
  ones16 = jnp.ones((CHUNK, 16), f32)

  d = _make_sc_deg()(dst_p, ones16, zeros16)

  grid = NPAD // _RB
  u3, dinv16 = pl.pallas_call(
      _tc_prescale_body,
      grid=(grid,),
      in_specs=[
          pl.BlockSpec((_RB, IN_CH), lambda i: (i, 0)),
          pl.BlockSpec((2, _RB, 16), lambda i: (0, i, 0)),
      ],
      out_specs=(pl.BlockSpec((2, _RB, 64), lambda i: (0, i, 0)),
                 pl.BlockSpec((_RB, 16), lambda i: (i, 0))),
      out_shape=(jax.ShapeDtypeStruct((2, NPAD, 64), f32),
                 jax.ShapeDtypeStruct((NPAD, 16), f32)),
  )(x, d)

  a = _make_sc_prop(True)(u3, src_sp, dst_sp, zeros64)

  u2 = pl.pallas_call(
      _tc_layer1_body,
      grid=(grid,),
      in_specs=[
          pl.BlockSpec((2, _RB, 64), lambda i: (0, i, 0)),
          pl.BlockSpec((2, _RB, 64), lambda i: (0, i, 0)),
          pl.BlockSpec((_RB, 16), lambda i: (i, 0)),
          pl.BlockSpec((IN_CH, HID), lambda i: (0, 0)),
          pl.BlockSpec((1, HID), lambda i: (0, 0)),
          pl.BlockSpec((HID, OUT_CH), lambda i: (0, 0)),
      ],
      out_specs=pl.BlockSpec((_RB, OUT_CH), lambda i: (i, 0)),
      out_shape=jax.ShapeDtypeStruct((NPAD, OUT_CH), f32),
  )(u3, a, dinv16, W1, b1r, W2)

  c = _make_sc_prop(False)(u2, src_p, dst_p, zeros64)

  out = pl.pallas_call(
      _tc_pool_body,
      grid=(grid,),
      in_specs=[
          pl.BlockSpec((_RB, OUT_CH), lambda i: (i, 0)),
          pl.BlockSpec((2, _RB, OUT_CH), lambda i: (0, i, 0)),
          pl.BlockSpec((_RB, 16), lambda i: (i, 0)),
          pl.BlockSpec((1, OUT_CH), lambda i: (0, 0)),
          pl.BlockSpec((_RB, 1), lambda i: (i, 0)),
      ],
      out_specs=pl.BlockSpec((G, OUT_CH), lambda i: (0, 0)),
      out_shape=jax.ShapeDtypeStruct((G, OUT_CH), f32),
      scratch_shapes=[pltpu.VMEM((G, 128), f32)],
  )(u2, c, dinv16, b2r, batch_p)
  return out


# speedup vs baseline: 1.0386x; 1.0234x over previous
import functools

import jax
import jax.numpy as jnp
from jax import lax
from jax.experimental import pallas as pl
from jax.experimental.pallas import tpu as pltpu
from jax.experimental.pallas import tpu_sc as plsc

N = 10000
NPAD = 10240
PT = NPAD // 16
E = 320000
CHUNK = 128
NW = 32
CHPW = 80
ECH = E // CHUNK
PCH = 60
EPAD = NW * CHPW * CHUNK
G = 64
IN_CH = 128
HID = 512
OUT_CH = 64


DEPTH = 4


def _make_sc_prop(split):
  C = 64
  CH = (EPAD // CHUNK) // 16 if split else CHPW
  mesh = plsc.VectorSubcoreMesh(core_axis_name="c", subcore_axis_name="s")

  nlast = 16 if split else NW
  rlast = ECH - (nlast - 1) * CH

  @functools.partial(
      pl.kernel,
      out_type=jax.ShapeDtypeStruct((2, NPAD, C), jnp.float32),
      mesh=mesh,
      scratch_types=[
          pltpu.VMEM_SHARED((NPAD, C), jnp.float32),
          pltpu.VMEM((CH, CHUNK), jnp.int32),
          pltpu.VMEM((CH, CHUNK), jnp.int32),
          [pltpu.VMEM((CHUNK, C), jnp.float32) for _ in range(DEPTH)],
          [pltpu.SemaphoreType.DMA for _ in range(DEPTH)],
          [pltpu.SemaphoreType.DMA for _ in range(DEPTH)],
      ],
      compiler_params=pltpu.CompilerParams(use_tc_tiling_on_sc=False),
  )
  def prop(u_hbm, src_hbm, dst_hbm, pad_hbm, zeros_hbm, out, acc_sh,
           sidx_a, didx_a, rows, sg, ss):
    cid = lax.axis_index("c")
    sid = lax.axis_index("s")
    if split:
      u_view = u_hbm.at[cid]
      w = sid
    else:
      u_view = u_hbm
      w = cid * 16 + sid

    @pl.when(w < nlast - 1)
    def _():
      pltpu.sync_copy(src_hbm.at[pl.ds(w * CH, CH)], sidx_a)
      pltpu.sync_copy(dst_hbm.at[pl.ds(w * CH, CH)], didx_a)

    @pl.when(w == nlast - 1)
    def _():
      pltpu.sync_copy(src_hbm.at[pl.ds(ECH - rlast, rlast)],
                      sidx_a.at[pl.ds(0, rlast)])
      pltpu.sync_copy(pad_hbm, sidx_a.at[pl.ds(rlast, PCH)])
      pltpu.sync_copy(dst_hbm.at[pl.ds(ECH - rlast, rlast)],
                      didx_a.at[pl.ds(0, rlast)])
      pltpu.sync_copy(pad_hbm, didx_a.at[pl.ds(rlast, PCH)])

    pltpu.sync_copy(zeros_hbm, acc_sh.at[pl.ds(sid * PT, PT)])
    plsc.subcore_barrier()

    for b in range(DEPTH - 1):
      pltpu.async_copy(u_view.at[sidx_a.at[b]], rows[b], sg[b])

    def body(i, carry):
      for b in range(DEPTH):
        k = i * DEPTH + b
        pltpu.make_async_copy(u_view.at[sidx_a.at[k]], rows[b], sg[b]).wait()

        bp = (b - 1) % DEPTH
        @pl.when(k >= 1)
        def _():
          pltpu.make_async_copy(
              rows[bp], acc_sh.at[didx_a.at[k]], ss[bp]).wait()

        @pl.when(k + DEPTH - 1 < CH)
        def _():
          pltpu.async_copy(u_view.at[sidx_a.at[k + DEPTH - 1]], rows[bp],
                           sg[bp])

        pltpu.async_copy(rows[b], acc_sh.at[didx_a.at[k]], ss[b], add=True)
      return carry

    lax.fori_loop(0, CH // DEPTH, body, 0)
    pltpu.make_async_copy(
        rows[(CH - 1) % DEPTH], acc_sh.at[didx_a.at[CH - 1]],
        ss[(CH - 1) % DEPTH]).wait()
    plsc.subcore_barrier()
    pltpu.sync_copy(acc_sh.at[pl.ds(sid * PT, PT)],
                    out.at[cid, pl.ds(sid * PT, PT)])

  return prop


def _make_sc_deg():
  mesh = plsc.VectorSubcoreMesh(core_axis_name="c", subcore_axis_name="s")
  C = 16

  @functools.partial(
      pl.kernel,
      out_type=jax.ShapeDtypeStruct((2, NPAD, C), jnp.float32),
      mesh=mesh,
      scratch_types=[
          pltpu.VMEM_SHARED((NPAD, C), jnp.float32),
          pltpu.VMEM((CHPW, CHUNK), jnp.int32),
          pltpu.VMEM((CHUNK, C), jnp.float32),
          pltpu.SemaphoreType.DMA,
      ],
      compiler_params=pltpu.CompilerParams(use_tc_tiling_on_sc=False),
  )
  def deg(dst_hbm, pad_hbm, ones_hbm, zeros_hbm, out, acc_sh, didx_a,
          ones_v, ss):
    cid = lax.axis_index("c")
    sid = lax.axis_index("s")
    w = cid * 16 + sid
    rlast = ECH - (NW - 1) * CHPW

    @pl.when(w < NW - 1)
    def _():
      pltpu.sync_copy(dst_hbm.at[pl.ds(w * CHPW, CHPW)], didx_a)

    @pl.when(w == NW - 1)
    def _():
      pltpu.sync_copy(dst_hbm.at[pl.ds(ECH - rlast, rlast)],
                      didx_a.at[pl.ds(0, rlast)])
      pltpu.sync_copy(pad_hbm, didx_a.at[pl.ds(rlast, PCH)])

    pltpu.sync_copy(zeros_hbm, acc_sh.at[pl.ds(sid * PT, PT)])
    pltpu.sync_copy(ones_hbm, ones_v)
    plsc.subcore_barrier()
    D = 8

    def body(k, carry):
      pltpu.async_copy(ones_v, acc_sh.at[didx_a.at[k]], ss, add=True)

      @pl.when(k >= D)
      def _():
        pltpu.make_async_copy(ones_v, acc_sh.at[didx_a.at[k]], ss).wait()

      return carry

    lax.fori_loop(0, CHPW, body, 0)
    for _ in range(D):
      pltpu.make_async_copy(ones_v, acc_sh.at[didx_a.at[0]], ss).wait()
    plsc.subcore_barrier()
    pltpu.sync_copy(acc_sh.at[pl.ds(sid * PT, PT)],
                    out.at[cid, pl.ds(sid * PT, PT)])

  return deg


_RB = 2048


def _tc_prescale_body(x_ref, d_ref, u_ref, dinv_ref):
  dinv = lax.rsqrt(1.0 + d_ref[0, :, :1] + d_ref[1, :, :1])
  u_ref[0] = x_ref[:, :64] * dinv
  u_ref[1] = x_ref[:, 64:] * dinv
  dinv_ref[...] = jnp.broadcast_to(dinv, dinv_ref.shape)


def _tc_layer1_body(u1_ref, a_ref, dinv_ref, w1_ref, b1_ref, w2_ref, u2_ref):
  i = pl.program_id(0)
  dinv = dinv_ref[:, :1]
  p1 = dinv * jnp.concatenate(
      [u1_ref[0] + a_ref[0], u1_ref[1] + a_ref[1]], axis=1)
  h1 = jnp.maximum(
      jnp.dot(p1, w1_ref[...], preferred_element_type=jnp.float32)
      + b1_ref[...], 0.0)
  t = jnp.dot(h1, w2_ref[...], preferred_element_type=jnp.float32)
  row = i * _RB + lax.broadcasted_iota(jnp.int32, (_RB, 1), 0)
  u2_ref[...] = jnp.where(row < N, dinv * t, 0.0)


def _tc_pool_body(u2_ref, c_ref, dinv_ref, b2_ref, batch_ref, o_ref, cnt_ref):
  i = pl.program_id(0)
  nsteps = pl.num_programs(0)
  dinv = dinv_ref[:, :1]
  p2 = dinv * (u2_ref[...] + c_ref[0] + c_ref[1])
  h2 = jnp.maximum(p2 + b2_ref[...], 0.0)
  row = i * _RB + lax.broadcasted_iota(jnp.int32, (_RB, 1), 0)
  h2 = jnp.where(row < N, h2, 0.0)
  m = (batch_ref[...] ==
       lax.broadcasted_iota(jnp.int32, (1, G), 1)).astype(jnp.float32)
  part = lax.dot_general(m, h2, (((0,), (0,)), ((), ())),
                         preferred_element_type=jnp.float32)
  pcnt = lax.dot_general(m, jnp.ones((_RB, 1), jnp.float32),
                         (((0,), (0,)), ((), ())),
                         preferred_element_type=jnp.float32)

  @pl.when(i == 0)
  def _():
    o_ref[...] = jnp.zeros_like(o_ref)
    cnt_ref[...] = jnp.zeros_like(cnt_ref)

  o_ref[...] += part
  cnt_ref[:, :1] += pcnt

  @pl.when(i == nsteps - 1)
  def _():
    o_ref[...] = o_ref[...] / jnp.maximum(cnt_ref[:, :1], 1.0)


def kernel(x, edge_index, batch, W1, b1, W2, b2):
  f32 = jnp.float32
  pad2 = (N + jnp.arange(PCH * CHUNK, dtype=jnp.int32) % (NPAD - N)
          ).reshape(PCH, CHUNK)
  e0 = edge_index[0].reshape(ECH, CHUNK)
  e1 = edge_index[1].reshape(ECH, CHUNK)
  batch_p = jnp.concatenate(
      [batch, jnp.full((NPAD - N,), G, jnp.int32)]).reshape(NPAD, 1)
  b1r = b1.reshape(1, HID)
  b2r = b2.reshape(1, OUT_CH)
  zeros16 = jnp.zeros((PT, 16), f32)
  zeros64 = jnp.zeros((PT, OUT_CH), f32)
  ones16 = jnp.ones((CHUNK, 16), f32)

  d = _make_sc_deg()(e1, pad2, ones16, zeros16)

  grid = NPAD // _RB
  u3, dinv16 = pl.pallas_call(
      _tc_prescale_body,
      grid=(grid,),
      in_specs=[
          pl.BlockSpec((_RB, IN_CH), lambda i: (i, 0)),
          pl.BlockSpec((2, _RB, 16), lambda i: (0, i, 0)),
      ],
      out_specs=(pl.BlockSpec((2, _RB, 64), lambda i: (0, i, 0)),
                 pl.BlockSpec((_RB, 16), lambda i: (i, 0))),
      out_shape=(jax.ShapeDtypeStruct((2, NPAD, 64), f32),
                 jax.ShapeDtypeStruct((NPAD, 16), f32)),
  )(x, d)

  a = _make_sc_prop(True)(u3, e0, e1, pad2, zeros64)

  u2 = pl.pallas_call(
      _tc_layer1_body,
      grid=(grid,),
      in_specs=[
          pl.BlockSpec((2, _RB, 64), lambda i: (0, i, 0)),
          pl.BlockSpec((2, _RB, 64), lambda i: (0, i, 0)),
          pl.BlockSpec((_RB, 16), lambda i: (i, 0)),
          pl.BlockSpec((IN_CH, HID), lambda i: (0, 0)),
          pl.BlockSpec((1, HID), lambda i: (0, 0)),
          pl.BlockSpec((HID, OUT_CH), lambda i: (0, 0)),
      ],
      out_specs=pl.BlockSpec((_RB, OUT_CH), lambda i: (i, 0)),
      out_shape=jax.ShapeDtypeStruct((NPAD, OUT_CH), f32),
  )(u3, a, dinv16, W1, b1r, W2)

  c = _make_sc_prop(False)(u2, e0, e1, pad2, zeros64)

  out = pl.pallas_call(
      _tc_pool_body,
      grid=(grid,),
      in_specs=[
          pl.BlockSpec((_RB, OUT_CH), lambda i: (i, 0)),
          pl.BlockSpec((2, _RB, OUT_CH), lambda i: (0, i, 0)),
          pl.BlockSpec((_RB, 16), lambda i: (i, 0)),
          pl.BlockSpec((1, OUT_CH), lambda i: (0, 0)),
          pl.BlockSpec((_RB, 1), lambda i: (i, 0)),
      ],
      out_specs=pl.BlockSpec((G, OUT_CH), lambda i: (0, 0)),
      out_shape=jax.ShapeDtypeStruct((G, OUT_CH), f32),
      scratch_shapes=[pltpu.VMEM((G, 128), f32)],
  )(u2, c, dinv16, b2r, batch_p)
  return out
